# baseline (device time: 12734 ns/iter reference)
import jax
import jax.numpy as jnp
from jax import lax
from jax.experimental import pallas as pl
from jax.experimental.pallas import tpu as pltpu

N_DEV = 4
EPS = 1e-5
OUT_CHUNKS = 2


def kernel(x, t_emb, W_scale, W_shift):
    b, s, c_loc = x.shape
    c_glob = c_loc * N_DEV
    s_chunk = s // OUT_CHUNKS

    def body(x_hbm, t_hbm, ws_hbm, wsh_hbm, out_hbm,
             xv, tv, wsv, wshv, outv, comm_ref,
             in_sems, out_sems, send_sems, recv_sems):
        my = lax.axis_index("i")

        cp_x = pltpu.make_async_copy(x_hbm, xv, in_sems.at[0])
        cp_t = pltpu.make_async_copy(t_hbm, tv, in_sems.at[1])
        cp_ws = pltpu.make_async_copy(ws_hbm, wsv, in_sems.at[2])
        cp_wsh = pltpu.make_async_copy(wsh_hbm, wshv, in_sems.at[3])
        cp_x.start()
        cp_t.start()
        cp_ws.start()
        cp_wsh.start()

        barrier_sem = pltpu.get_barrier_semaphore()
        for d in range(1, N_DEV):
            pl.semaphore_signal(
                barrier_sem, inc=1,
                device_id=((my + d) % N_DEV,),
                device_id_type=pl.DeviceIdType.MESH,
            )
        pl.semaphore_wait(barrier_sem, N_DEV - 1)

        cp_x.wait()
        xf = xv[...]
        psum = jnp.sum(xf, axis=-1)
        psumsq = jnp.sum(xf * xf, axis=-1)
        comm_ref[0] = jnp.concatenate([psum, psumsq], axis=0)

        rdmas = []
        for d in range(1, N_DEV):
            rdma = pltpu.make_async_remote_copy(
                src_ref=comm_ref.at[0],
                dst_ref=comm_ref.at[d],
                send_sem=send_sems.at[d - 1],
                recv_sem=recv_sems.at[d - 1],
                device_id=((my + d) % N_DEV,),
                device_id_type=pl.DeviceIdType.MESH,
            )
            rdma.start()
            rdmas.append(rdma)

        cp_t.wait()
        cp_ws.wait()
        cp_wsh.wait()
        scale = jnp.dot(tv[...], wsv[...],
                        preferred_element_type=jnp.float32)
        shift = jnp.dot(tv[...], wshv[...],
                        preferred_element_type=jnp.float32)

        for rdma in rdmas:
            rdma.wait()

        total = comm_ref[0] + comm_ref[1] + comm_ref[2] + comm_ref[3]
        mean = total[:b] / c_glob
        meansq = total[b:] / c_glob
        var = meansq - mean * mean
        rstd = lax.rsqrt(var + EPS)

        out_cps = []
        for k in range(OUT_CHUNKS):
            sl = slice(k * s_chunk, (k + 1) * s_chunk)
            h = (xf[:, sl, :] - mean[:, sl, None]) * rstd[:, sl, None]
            outv[:, sl, :] = (h * (1.0 + scale[:, None, :])
                              + shift[:, None, :]).astype(outv.dtype)
            cp = pltpu.make_async_copy(
                outv.at[:, sl, :], out_hbm.at[:, sl, :], out_sems.at[k])
            cp.start()
            out_cps.append(cp)
        for cp in out_cps:
            cp.wait()

    return pl.pallas_call(
        body,
        out_shape=jax.ShapeDtypeStruct((b, s, c_loc), jnp.float32),
        in_specs=[pl.BlockSpec(memory_space=pltpu.MemorySpace.HBM)] * 4,
        out_specs=pl.BlockSpec(memory_space=pltpu.MemorySpace.HBM),
        scratch_shapes=[
            pltpu.VMEM((b, s, c_loc), jnp.float32),
            pltpu.VMEM(t_emb.shape, jnp.float32),
            pltpu.VMEM(W_scale.shape, jnp.float32),
            pltpu.VMEM(W_shift.shape, jnp.float32),
            pltpu.VMEM((b, s, c_loc), jnp.float32),
            pltpu.VMEM((N_DEV, 2 * b, s), jnp.float32),
            pltpu.SemaphoreType.DMA((4,)),
            pltpu.SemaphoreType.DMA((OUT_CHUNKS,)),
            pltpu.SemaphoreType.DMA((N_DEV - 1,)),
            pltpu.SemaphoreType.DMA((N_DEV - 1,)),
        ],
        compiler_params=pltpu.CompilerParams(collective_id=0),
    )(x, t_emb, W_scale, W_shift)


# device time: 9208 ns/iter; 1.3829x vs baseline; 1.3829x over previous
import jax
import jax.numpy as jnp
from jax import lax
from jax.experimental import pallas as pl
from jax.experimental.pallas import tpu as pltpu

N_DEV = 4
EPS = 1e-5
OUT_CHUNKS = 2


def kernel(x, t_emb, W_scale, W_shift):
    b, s, c_loc = x.shape
    c_glob = c_loc * N_DEV
    s_chunk = s // OUT_CHUNKS

    def body(x_hbm, t_hbm, ws_hbm, wsh_hbm, out_hbm,
             xv, tv, wsv, wshv, outv, comm_ref,
             in_sems, out_sems, send_sems, recv_sems):
        my = lax.axis_index("i")

        cp_x = pltpu.make_async_copy(x_hbm, xv, in_sems.at[0])
        cp_t = pltpu.make_async_copy(t_hbm, tv, in_sems.at[1])
        cp_ws = pltpu.make_async_copy(ws_hbm, wsv, in_sems.at[2])
        cp_wsh = pltpu.make_async_copy(wsh_hbm, wshv, in_sems.at[3])
        cp_x.start()
        cp_t.start()
        cp_ws.start()
        cp_wsh.start()

        barrier_sem = pltpu.get_barrier_semaphore()
        for d in range(1, N_DEV):
            pl.semaphore_signal(
                barrier_sem, inc=1,
                device_id=((my + d) % N_DEV,),
                device_id_type=pl.DeviceIdType.MESH,
            )
        pl.semaphore_wait(barrier_sem, N_DEV - 1)

        cp_x.wait()
        xf = xv[...]
        psum = jnp.sum(xf, axis=-1)
        psumsq = jnp.sum(xf * xf, axis=-1)
        comm_ref[0] = jnp.concatenate([psum, psumsq], axis=0)

        rdmas = []
        for d in range(1, N_DEV):
            rdma = pltpu.make_async_remote_copy(
                src_ref=comm_ref.at[0],
                dst_ref=comm_ref.at[d],
                send_sem=send_sems.at[d - 1],
                recv_sem=recv_sems.at[d - 1],
                device_id=((my + d) % N_DEV,),
                device_id_type=pl.DeviceIdType.MESH,
            )
            rdma.start()
            rdmas.append(rdma)

        cp_t.wait()
        cp_ws.wait()
        cp_wsh.wait()
        scale = jnp.dot(tv[...], wsv[...],
                        preferred_element_type=jnp.float32)
        shift = jnp.dot(tv[...], wshv[...],
                        preferred_element_type=jnp.float32)

        for rdma in rdmas:
            rdma.wait()

        total = comm_ref[0] + comm_ref[1] + comm_ref[2] + comm_ref[3]
        mean = total[:b] / c_glob
        meansq = total[b:] / c_glob
        var = meansq - mean * mean
        rstd = lax.rsqrt(var + EPS)

        out_cps = []
        for k in range(OUT_CHUNKS):
            sl = slice(k * s_chunk, (k + 1) * s_chunk)
            h = (xf[:, sl, :] - mean[:, sl, None]) * rstd[:, sl, None]
            outv[:, sl, :] = (h * (1.0 + scale[:, None, :])
                              + shift[:, None, :]).astype(outv.dtype)
            cp = pltpu.make_async_copy(
                outv.at[:, sl, :], out_hbm.at[:, sl, :], out_sems.at[k])
            cp.start()
            out_cps.append(cp)
        for cp in out_cps:
            cp.wait()

    hbm = pltpu.MemorySpace.HBM
    x = pltpu.with_memory_space_constraint(x, hbm)
    t_emb = pltpu.with_memory_space_constraint(t_emb, hbm)
    W_scale = pltpu.with_memory_space_constraint(W_scale, hbm)
    W_shift = pltpu.with_memory_space_constraint(W_shift, hbm)
    out = pl.pallas_call(
        body,
        out_shape=jax.ShapeDtypeStruct((b, s, c_loc), jnp.float32),
        in_specs=[pl.BlockSpec(memory_space=pltpu.MemorySpace.HBM)] * 4,
        out_specs=pl.BlockSpec(memory_space=pltpu.MemorySpace.HBM),
        scratch_shapes=[
            pltpu.VMEM((b, s, c_loc), jnp.float32),
            pltpu.VMEM(t_emb.shape, jnp.float32),
            pltpu.VMEM(W_scale.shape, jnp.float32),
            pltpu.VMEM(W_shift.shape, jnp.float32),
            pltpu.VMEM((b, s, c_loc), jnp.float32),
            pltpu.VMEM((N_DEV, 2 * b, s), jnp.float32),
            pltpu.SemaphoreType.DMA((4,)),
            pltpu.SemaphoreType.DMA((OUT_CHUNKS,)),
            pltpu.SemaphoreType.DMA((N_DEV - 1,)),
            pltpu.SemaphoreType.DMA((N_DEV - 1,)),
        ],
        compiler_params=pltpu.CompilerParams(collective_id=0),
    )(x, t_emb, W_scale, W_shift)
    return out
